# Initial kernel scaffold; baseline (speedup 1.0000x reference)
import jax
import jax.numpy as jnp
from jax.experimental import pallas as pl


def kernel(X, Fmat, B, bn_weight, bn_bias, edge_index):
    raise NotImplementedError("write your pallas kernel here")


# speedup vs baseline: 1.6928x; 1.6928x over previous
import functools

import jax
import jax.numpy as jnp
import numpy as np
from jax import lax
from jax.experimental import pallas as pl
from jax.experimental.pallas import tpu as pltpu
from jax.experimental.pallas import tpu_sc as plsc

N = 10000
E = 320000
M = 128
MY = 16
K = 10
GAMMA = 0.8
EPS_F = 1e-12

NC = 2
NS = 16
NW = NC * NS
CS = 128
CHW = 79
PE_CAP = CHW * CS
NP = 10240
TROWS = NP // NS
WINSZ = 64
SEGWIN = 16
SEGCAP = SEGWIN * WINSZ
TRASH = N + 8
SIDE0 = N + 16

_SIZES_SC = [10080] * 11 + [9840] * 4 + [9760]
_SIZES = np.array(_SIZES_SC * NC, np.int64)
_STARTS = np.concatenate([[0], np.cumsum(_SIZES)])
assert _STARTS[-1] == E


def _sc_body(xt, rows3, flags3, sids3, out, agg_sh, rbuf, fbuf, sbuf, gbuf,
             stage, gsem):
    c = lax.axis_index("c")
    s = lax.axis_index("s")
    wid = c * NS + s

    pltpu.sync_copy(sids3.at[wid], sbuf)

    z16 = jnp.zeros((16,), jnp.float32)

    def zrow(i, carry):
        for jj in range(8):
            gbuf[i, pl.ds(jj * 16, 16)] = z16
        return carry

    lax.fori_loop(0, CS, zrow, 0)
    zbase = s * TROWS
    for t in range(TROWS // CS):
        pltpu.sync_copy(gbuf, agg_sh.at[pl.ds(zbase + t * CS, CS)])
    plsc.subcore_barrier()

    def chunk(cidx, carry):
        pltpu.sync_copy(rows3.at[wid, cidx], rbuf)
        pltpu.sync_copy(flags3.at[wid, cidx], fbuf)
        pltpu.async_copy(xt.at[rbuf], gbuf, gsem).wait()

        def edge16(j16, ecarry):
            si = ecarry[0]
            accs = list(ecarry[1:])
            fv = fbuf[pl.ds(j16 * 16, 16)]
            for jj in range(16):
                f = fv[jj]
                si = si + f
                srow = lax.rem(si, 3 * WINSZ)
                keep = (1 - f).astype(jnp.float32)
                j = j16 * 16 + jj
                for i in range(8):
                    v = gbuf[j, pl.ds(i * 16, 16)]
                    a = accs[i] * keep + v
                    stage[srow, pl.ds(i * 16, 16)] = a
                    accs[i] = a
            return (si, *accs)

        carry = lax.fori_loop(0, CS // 16, edge16, carry)
        si = carry[0]
        wcur = jnp.minimum(lax.shift_right_logical(si, 6), SEGWIN - 1)
        for back in (2, 1, 0):
            wk = jnp.maximum(wcur - back, 0)
            pltpu.sync_copy(
                stage.at[pl.ds(lax.rem(wk, 3) * WINSZ, WINSZ)],
                agg_sh.at[sbuf.at[wk]])
        return carry

    init = (jnp.int32(-1),) + tuple(z16 for _ in range(8))
    lax.fori_loop(0, CHW, chunk, init)
    plsc.subcore_barrier()

    for t in range(TROWS // CS):
        off = zbase + t * CS
        pltpu.sync_copy(agg_sh.at[pl.ds(off, CS)], gbuf)
        pltpu.sync_copy(gbuf, out.at[c, pl.ds(off, CS)])


@functools.cache
def _sc_propagate():
    mesh = plsc.VectorSubcoreMesh(
        core_axis_name="c", subcore_axis_name="s",
        num_cores=NC, num_subcores=NS)
    return pl.kernel(
        _sc_body,
        out_type=jax.ShapeDtypeStruct((NC, NP, M), jnp.float32),
        mesh=mesh,
        scratch_types=[
            pltpu.VMEM_SHARED((NP, M), jnp.float32),
            pltpu.VMEM((CS,), jnp.int32),
            pltpu.VMEM((CS,), jnp.int32),
            pltpu.VMEM((SEGWIN, WINSZ), jnp.int32),
            pltpu.VMEM((CS, M), jnp.float32),
            pltpu.VMEM((3 * WINSZ, M), jnp.float32),
            pltpu.SemaphoreType.DMA,
        ],
    )


def _g_body(f_ref, g_ref):
    f = f_ref[...]
    ff = lax.dot_general(f, f, (((0,), (0,)), ((), ())))
    nrm = jnp.sqrt(jnp.sum(ff * ff))
    g_ref[...] = ff / (nrm + EPS_F)


def _merge(fni_ref, parts_ref, acc_ref):
    acc_ref[...] = parts_ref[0] + parts_ref[1]
    for w in range(NW):
        r = fni_ref[w]
        acc_ref[pl.ds(r, 1), :] = (acc_ref[pl.ds(r, 1), :]
                                   + acc_ref[pl.ds(SIDE0 + w, 1), :])


def _update_body(fni_ref, parts_ref, xt0_ref, g_ref, out_ref, acc_ref):
    _merge(fni_ref, parts_ref, acc_ref)
    prop = lax.dot_general(acc_ref[...], g_ref[...], (((1,), (1,)), ((), ())))
    out_ref[...] = GAMMA * prop + xt0_ref[...]


def _final_body(fni_ref, parts_ref, xt0_ref, g_ref, b_ref, w_ref, bias_ref,
                out_ref, acc_ref):
    _merge(fni_ref, parts_ref, acc_ref)
    agg = acc_ref[:N]
    prop = lax.dot_general(agg, g_ref[...], (((1,), (1,)), ((), ())))
    xt = GAMMA * prop + xt0_ref[:N]
    mean = jnp.mean(xt, axis=0, keepdims=True)
    xc = xt - mean
    var = jnp.mean(xc * xc, axis=0, keepdims=True)
    y = xc * (w_ref[...] / jnp.sqrt(var + 1e-5)) + bias_ref[...]
    out_ref[...] = lax.dot_general(y, b_ref[...], (((1,), (1,)), ((), ())))


_SMEM = pl.BlockSpec(memory_space=pltpu.SMEM)
_ANY = pl.BlockSpec(memory_space=pltpu.ANY) if hasattr(pltpu, "ANY") else None

_update_call = pl.pallas_call(
    _update_body,
    out_shape=jax.ShapeDtypeStruct((NP, M), jnp.float32),
    in_specs=[_SMEM, pl.BlockSpec(), pl.BlockSpec(), pl.BlockSpec()],
    scratch_shapes=[pltpu.VMEM((NP, M), jnp.float32)],
)

_final_call = pl.pallas_call(
    _final_body,
    out_shape=jax.ShapeDtypeStruct((N, MY), jnp.float32),
    in_specs=[_SMEM] + [pl.BlockSpec()] * 6,
    scratch_shapes=[pltpu.VMEM((NP, M), jnp.float32)],
)

_g_call = pl.pallas_call(
    _g_body, out_shape=jax.ShapeDtypeStruct((M, M), jnp.float32))


def kernel(X, Fmat, B, bn_weight, bn_bias, edge_index):
    xt0 = jnp.concatenate(
        [X.T, jnp.zeros((NP - N, M), jnp.float32)])
    row = edge_index[0]
    col = edge_index[1]

    order = jnp.argsort(col, stable=True)
    col_s = col[order]
    row_s = row[order]
    starts_j = jnp.asarray(_STARTS[:NW], dtype=jnp.int32)
    sizes_j = jnp.asarray(_SIZES, dtype=jnp.int32)
    offs = starts_j[:, None] + jnp.arange(PE_CAP, dtype=jnp.int32)[None, :]
    valid = jnp.arange(PE_CAP, dtype=jnp.int32)[None, :] < sizes_j[:, None]
    offc = jnp.minimum(offs, E - 1)
    rows_w = jnp.where(valid, row_s[offc], 0).astype(jnp.int32)
    cols_w = jnp.where(valid, col_s[offc], TRASH).astype(jnp.int32)
    prevc = jnp.concatenate(
        [jnp.full((NW, 1), -1, jnp.int32), cols_w[:, :-1]], axis=1)
    flags = (cols_w != prevc).astype(jnp.int32)
    segi = jnp.minimum(jnp.cumsum(flags, axis=1) - 1, SEGCAP - 1)
    wi = jnp.broadcast_to(jnp.arange(NW, dtype=jnp.int32)[:, None], segi.shape)
    tgt = jnp.where(flags == 1, segi, SEGCAP - 1)
    sidv = jnp.where(flags == 1, cols_w, TRASH)
    sids = jnp.full((NW, SEGCAP), TRASH, jnp.int32).at[wi, tgt].set(sidv)
    sids = sids.at[:, 0].set(SIDE0 + jnp.arange(NW, dtype=jnp.int32))
    fni = col_s[starts_j].astype(jnp.int32)

    rows3 = rows_w.reshape(NW, CHW, CS)
    flags3 = flags.reshape(NW, CHW, CS)
    sids3 = sids.reshape(NW, SEGWIN, WINSZ)

    g = _g_call(Fmat)
    w2 = bn_weight.reshape(1, M)
    b2 = bn_bias.reshape(1, M)

    prop = _sc_propagate()
    xt = xt0
    for _ in range(K - 1):
        parts = prop(xt, rows3, flags3, sids3)
        xt = _update_call(fni, parts, xt0, g)
    parts = prop(xt, rows3, flags3, sids3)
    return _final_call(fni, parts, xt0, g, B, w2, b2)


# speedup vs baseline: 2.1714x; 1.2827x over previous
import functools

import jax
import jax.numpy as jnp
import numpy as np
from jax import lax
from jax.experimental import pallas as pl
from jax.experimental.pallas import tpu as pltpu
from jax.experimental.pallas import tpu_sc as plsc

N = 10000
E = 320000
M = 128
MY = 16
K = 10
GAMMA = 0.8
EPS_F = 1e-12

NC = 2
NS = 16
NW = NC * NS
CS = 96
CHW = 105
PE_CAP = CHW * CS
NPX = 10048
WINSZ = 64
SEGWIN = 16
SEGCAP = SEGWIN * WINSZ
STROWS = 256
TRASH = N + 8
SIDE0 = N + 16

_SIZES_SC = [10080] * 11 + [9840] * 4 + [9760]
_SIZES = np.array(_SIZES_SC * NC, np.int64)
_STARTS = np.concatenate([[0], np.cumsum(_SIZES)])
assert _STARTS[-1] == E


def _sc_body(xt, rows3, slots3, sids3, out, rbuf, sltbuf, sbuf, gbuf, stage,
             gsem):
    c = lax.axis_index("c")
    s = lax.axis_index("s")
    wid = c * NS + s

    pltpu.sync_copy(sids3.at[wid], sbuf)

    z16 = jnp.zeros((16,), jnp.float32)

    def zrow(i, carry):
        for jj in range(8):
            stage[i, pl.ds(jj * 16, 16)] = z16
        return carry

    lax.fori_loop(0, STROWS, zrow, 0)

    def publish(k, carry):
        pltpu.sync_copy(
            stage.at[pl.ds(jnp.bitwise_and(k, 3) * WINSZ, WINSZ)],
            out.at[sbuf.at[k]])
        base = jnp.bitwise_and(k, 3) * WINSZ

        def zw(i, cc):
            for jj in range(8):
                stage[base + i, pl.ds(jj * 16, 16)] = z16
            return cc

        lax.fori_loop(0, WINSZ, zw, 0)
        return carry

    def chunk(cidx, carry):
        prev = carry[0]
        wpub = carry[1]
        accs = list(carry[2:])
        pltpu.sync_copy(rows3.at[wid, cidx], rbuf)
        pltpu.sync_copy(slots3.at[wid, cidx], sltbuf)
        pltpu.async_copy(xt.at[rbuf], gbuf, gsem).wait()

        def edge16(j16, ecarry):
            prev = ecarry[0]
            accs = list(ecarry[1:])
            sv = sltbuf[pl.ds(j16 * 16, 16)]
            for jj in range(16):
                sl = sv[jj]
                keep = (sl == prev).astype(jnp.float32)
                srow = jnp.bitwise_and(sl, STROWS - 1)
                j = j16 * 16 + jj
                for i in range(8):
                    v = gbuf[j, pl.ds(i * 16, 16)]
                    a = accs[i] * keep + v
                    stage[srow, pl.ds(i * 16, 16)] = a
                    accs[i] = a
                prev = sl
            return (prev, *accs)

        ec = lax.fori_loop(0, CS // 16, edge16, (prev, *accs))
        prev = ec[0]
        accs = list(ec[1:])
        wcur = lax.shift_right_logical(prev, 6)
        lax.fori_loop(wpub, wcur, publish, 0)
        return (prev, jnp.maximum(wpub, wcur), *accs)

    init = (jnp.int32(-1), jnp.int32(0)) + tuple(z16 for _ in range(8))
    fin = lax.fori_loop(0, CHW, chunk, init)
    wlast = jnp.minimum(lax.shift_right_logical(fin[0], 6), SEGWIN - 1)
    lax.fori_loop(fin[1], wlast + 1, publish, 0)


@functools.cache
def _sc_propagate():
    mesh = plsc.VectorSubcoreMesh(
        core_axis_name="c", subcore_axis_name="s",
        num_cores=NC, num_subcores=NS)
    return pl.kernel(
        _sc_body,
        out_type=jax.ShapeDtypeStruct((NPX, M), jnp.float32),
        mesh=mesh,
        scratch_types=[
            pltpu.VMEM((CS,), jnp.int32),
            pltpu.VMEM((CS,), jnp.int32),
            pltpu.VMEM((SEGWIN, WINSZ), jnp.int32),
            pltpu.VMEM((CS, M), jnp.float32),
            pltpu.VMEM((STROWS, M), jnp.float32),
            pltpu.SemaphoreType.DMA,
        ],
    )


def _g_body(f_ref, g_ref):
    f = f_ref[...]
    ff = lax.dot_general(f, f, (((0,), (0,)), ((), ())))
    nrm = jnp.sqrt(jnp.sum(ff * ff))
    g_ref[...] = ff / (nrm + EPS_F)


def _merge(fni_ref, parts_ref, acc_ref):
    acc_ref[...] = parts_ref[...]
    for w in range(NW):
        r = fni_ref[w]
        acc_ref[pl.ds(r, 1), :] = (acc_ref[pl.ds(r, 1), :]
                                   + acc_ref[pl.ds(SIDE0 + w, 1), :])


def _update_body(fni_ref, parts_ref, xt0_ref, g_ref, out_ref, acc_ref):
    _merge(fni_ref, parts_ref, acc_ref)
    prop = lax.dot_general(acc_ref[...], g_ref[...], (((1,), (1,)), ((), ())))
    out_ref[...] = GAMMA * prop + xt0_ref[...]


def _final_body(fni_ref, parts_ref, xt0_ref, g_ref, b_ref, w_ref, bias_ref,
                out_ref, acc_ref):
    _merge(fni_ref, parts_ref, acc_ref)
    agg = acc_ref[:N]
    prop = lax.dot_general(agg, g_ref[...], (((1,), (1,)), ((), ())))
    xt = GAMMA * prop + xt0_ref[:N]
    mean = jnp.mean(xt, axis=0, keepdims=True)
    xc = xt - mean
    var = jnp.mean(xc * xc, axis=0, keepdims=True)
    y = xc * (w_ref[...] / jnp.sqrt(var + 1e-5)) + bias_ref[...]
    out_ref[...] = lax.dot_general(y, b_ref[...], (((1,), (1,)), ((), ())))


_SMEM = pl.BlockSpec(memory_space=pltpu.SMEM)
_ANY = pl.BlockSpec(memory_space=pltpu.ANY) if hasattr(pltpu, "ANY") else None

_update_call = pl.pallas_call(
    _update_body,
    out_shape=jax.ShapeDtypeStruct((NPX, M), jnp.float32),
    in_specs=[_SMEM, pl.BlockSpec(), pl.BlockSpec(), pl.BlockSpec()],
    scratch_shapes=[pltpu.VMEM((NPX, M), jnp.float32)],
)

_final_call = pl.pallas_call(
    _final_body,
    out_shape=jax.ShapeDtypeStruct((N, MY), jnp.float32),
    in_specs=[_SMEM] + [pl.BlockSpec()] * 6,
    scratch_shapes=[pltpu.VMEM((NPX, M), jnp.float32)],
)

_g_call = pl.pallas_call(
    _g_body, out_shape=jax.ShapeDtypeStruct((M, M), jnp.float32))


def kernel(X, Fmat, B, bn_weight, bn_bias, edge_index):
    xt0 = jnp.concatenate(
        [X.T, jnp.zeros((NPX - N, M), jnp.float32)])
    row = edge_index[0]
    col = edge_index[1]

    order = jnp.argsort(col, stable=True)
    col_s = col[order]
    row_s = row[order]
    starts_j = jnp.asarray(_STARTS[:NW], dtype=jnp.int32)
    sizes_j = jnp.asarray(_SIZES, dtype=jnp.int32)
    offs = starts_j[:, None] + jnp.arange(PE_CAP, dtype=jnp.int32)[None, :]
    valid = jnp.arange(PE_CAP, dtype=jnp.int32)[None, :] < sizes_j[:, None]
    offc = jnp.minimum(offs, E - 1)
    rows_w = jnp.where(valid, row_s[offc], 0).astype(jnp.int32)
    cols_w = jnp.where(valid, col_s[offc], TRASH).astype(jnp.int32)
    fni = col_s[starts_j].astype(jnp.int32)
    nb = fni.at[0].set(-1)
    ce = jnp.concatenate([fni[1:], jnp.array([N - 1], jnp.int32)])
    prevc = jnp.concatenate(
        [jnp.full((NW, 1), -1, jnp.int32), cols_w[:, :-1]], axis=1)
    flags = (cols_w != prevc).astype(jnp.int32)
    segi = jnp.cumsum(flags, axis=1) - 1
    slotv = cols_w - nb[:, None]
    slotv = jnp.where(segi == 0, 0, slotv)
    slotv = jnp.where(cols_w == TRASH, (ce - nb)[:, None] + 1, slotv)
    slotv = jnp.clip(slotv, 0, SEGCAP - 1).astype(jnp.int32)
    p = jnp.arange(SEGCAP, dtype=jnp.int32)[None, :]
    nodep = nb[:, None] + p
    sids = jnp.where(nodep <= ce[:, None], nodep, TRASH)
    sids = sids.at[:, 0].set(SIDE0 + jnp.arange(NW, dtype=jnp.int32))
    sids = sids.astype(jnp.int32)

    rows3 = rows_w.reshape(NW, CHW, CS)
    slots3 = slotv.reshape(NW, CHW, CS)
    sids3 = sids.reshape(NW, SEGWIN, WINSZ)

    g = _g_call(Fmat)
    w2 = bn_weight.reshape(1, M)
    b2 = bn_bias.reshape(1, M)

    prop = _sc_propagate()
    xt = xt0
    for _ in range(K - 1):
        parts = prop(xt, rows3, slots3, sids3)
        xt = _update_call(fni, parts, xt0, g)
    parts = prop(xt, rows3, slots3, sids3)
    return _final_call(fni, parts, xt0, g, B, w2, b2)
